# Initial kernel scaffold; baseline (speedup 1.0000x reference)
import functools

import jax
import jax.numpy as jnp
from jax import lax
from jax.experimental import pallas as pl
from jax.experimental.pallas import tpu as pltpu
from jax.experimental.pallas import tpu_sc as plsc

N = 10000
NP = 10240
E = 320000
PAD1 = 1536
ET = E + PAD1 + NP
IN_CH = 128
C = 32
DUMMY = N

NWORK = 32
EPW = ET // NWORK
CHUNK = 128
NCH = EPW // CHUNK
ZROWS = NP // 16

EBLK = 2048
NEBLK = ET // EBLK
NBLK = 1280
NNBLK = NP // NBLK

_mesh = plsc.VectorSubcoreMesh(core_axis_name="c", subcore_axis_name="s")


@functools.partial(
    pl.kernel,
    mesh=_mesh,
    out_type=[
        jax.ShapeDtypeStruct((ET, C), jnp.float32),
        jax.ShapeDtypeStruct((ET, C), jnp.float32),
        jax.ShapeDtypeStruct((2, NP, 16), jnp.float32),
    ],
    scratch_types=[
        pltpu.VMEM((CHUNK,), jnp.int32),
        pltpu.VMEM((CHUNK,), jnp.int32),
        pltpu.VMEM((CHUNK,), jnp.int32),
        pltpu.VMEM((CHUNK,), jnp.float32),
        pltpu.VMEM((CHUNK, C), jnp.float32),
        pltpu.VMEM((CHUNK, C), jnp.float32),
        pltpu.VMEM((CHUNK, 16), jnp.float32),
        pltpu.VMEM((ZROWS, 16), jnp.float32),
        pltpu.VMEM_SHARED((NP, 16), jnp.float32),
        pltpu.SemaphoreType.DMA,
        pltpu.SemaphoreType.DMA,
    ],
)
def _sc_gather_wstats(srcg_hbm, dstg_hbm, dstw_hbm, ew_hbm, xl_hbm, xr_hbm,
                      XL_hbm, XR_hbm, WACC_hbm,
                      isrc, idstg, idstw, vew, rows_l, rows_r, w16, zbuf,
                      wacc_sh, sem1, sem2):
    cid = lax.axis_index("c")
    sid = lax.axis_index("s")
    wid = sid * 2 + cid
    zero16 = jnp.zeros((16,), jnp.float32)

    @pl.loop(0, ZROWS)
    def _(r):
        zbuf[r, :] = zero16

    @pl.loop(0, CHUNK)
    def _(r):
        w16[r, :] = zero16

    pltpu.sync_copy(zbuf, wacc_sh.at[pl.ds(sid * ZROWS, ZROWS)])
    plsc.subcore_barrier()

    lane = lax.iota(jnp.int32, 16)
    col0 = jnp.zeros((16,), jnp.int32)
    col1 = jnp.full((16,), 1, jnp.int32)
    ones16 = jnp.full((16,), 1.0, jnp.float32)

    @pl.loop(0, NCH)
    def _(j):
        base = wid * EPW + j * CHUNK
        pltpu.sync_copy(srcg_hbm.at[pl.ds(base, CHUNK)], isrc)
        pltpu.sync_copy(dstg_hbm.at[pl.ds(base, CHUNK)], idstg)
        pltpu.sync_copy(dstw_hbm.at[pl.ds(base, CHUNK)], idstw)
        pltpu.sync_copy(ew_hbm.at[pl.ds(base, CHUNK)], vew)
        cl = pltpu.async_copy(xl_hbm.at[isrc], rows_l, sem1)
        cr = pltpu.async_copy(xr_hbm.at[idstg], rows_r, sem2)
        for k in range(CHUNK // 16):
            ridx = lane + (k * 16)
            plsc.store_scatter(w16, [ridx, col0], vew[pl.ds(k * 16, 16)])
            plsc.store_scatter(w16, [ridx, col1], ones16)
        pltpu.sync_copy(w16, wacc_sh.at[idstw], add=True)
        cl.wait()
        cr.wait()
        pltpu.sync_copy(rows_l, XL_hbm.at[pl.ds(base, CHUNK)])
        pltpu.sync_copy(rows_r, XR_hbm.at[pl.ds(base, CHUNK)])

    plsc.subcore_barrier()
    pltpu.sync_copy(wacc_sh.at[pl.ds(sid * ZROWS, ZROWS)],
                    WACC_hbm.at[cid, pl.ds(sid * ZROWS, ZROWS)])


@functools.partial(
    pl.kernel,
    mesh=_mesh,
    out_type=jax.ShapeDtypeStruct((2, NP, 48), jnp.float32),
    scratch_types=[
        pltpu.VMEM((CHUNK,), jnp.int32),
        pltpu.VMEM((CHUNK, 48), jnp.float32),
        pltpu.VMEM((ZROWS, 48), jnp.float32),
        pltpu.VMEM_SHARED((NP, 48), jnp.float32),
        pltpu.SemaphoreType.DMA,
    ],
)
def _sc_scatter_msgs(msg_hbm, dsts_hbm, ACC_hbm, idx, mbuf, zbuf, acc_sh, sem):
    cid = lax.axis_index("c")
    sid = lax.axis_index("s")
    wid = sid * 2 + cid
    zero16 = jnp.zeros((16,), jnp.float32)

    @pl.loop(0, ZROWS)
    def _(r):
        for k in range(3):
            zbuf[r, pl.ds(k * 16, 16)] = zero16

    pltpu.sync_copy(zbuf, acc_sh.at[pl.ds(sid * ZROWS, ZROWS)])
    plsc.subcore_barrier()

    @pl.loop(0, NCH)
    def _(j):
        base = wid * EPW + j * CHUNK
        pltpu.sync_copy(dsts_hbm.at[pl.ds(base, CHUNK)], idx)
        pltpu.sync_copy(msg_hbm.at[pl.ds(base, CHUNK)], mbuf)
        pltpu.sync_copy(mbuf, acc_sh.at[idx], add=True)

    plsc.subcore_barrier()
    pltpu.sync_copy(acc_sh.at[pl.ds(sid * ZROWS, ZROWS)],
                    ACC_hbm.at[cid, pl.ds(sid * ZROWS, ZROWS)])


def _proj_body(x_ref, wl_ref, bl_ref, wr_ref, br_ref, xl_ref, xr_ref):
    xb = x_ref[...]
    xl_ref[...] = jnp.dot(xb, wl_ref[...],
                          preferred_element_type=jnp.float32) + bl_ref[...]
    xr_ref[...] = jnp.dot(xb, wr_ref[...],
                          preferred_element_type=jnp.float32) + br_ref[...]


def _la_body(wacc_ref, la_ref):
    w = wacc_ref[0] + wacc_ref[1]
    la_ref[...] = w[:, 0:1] / jnp.maximum(w[:, 1:2], 1.0)


def _alpha_body(xl_ref, xr_ref, attr_ref, we_ref, att_ref, alpha_ref, pmax_ref):
    m = xl_ref[...] + xr_ref[...] + attr_ref[...] * we_ref[...]
    m = jnp.where(m >= 0.0, m, 0.2 * m)
    a = jnp.sum(m * att_ref[...], axis=1, keepdims=True)
    alpha_ref[...] = a
    pmax_ref[...] = jnp.broadcast_to(jnp.max(a), (1, 1, 128))


def _msg_body(alpha_ref, xl_ref, g_ref, msg_ref):
    ex = jnp.exp(alpha_ref[...] - g_ref[...])
    msg_ref[...] = jnp.concatenate(
        [ex * xl_ref[...], jnp.broadcast_to(ex, (EBLK, 16))], axis=1)


def _final_body(acc_ref, bias_ref, out_ref):
    a = acc_ref[0] + acc_ref[1]
    o = a[:, :C] / (a[:, C:C + 1] + 1e-16) + bias_ref[...]
    out_ref[...] = jnp.where(o >= 0.0, o, 0.01 * o)


def kernel(x, edge_index, edge_weight, W_l, b_l, W_r, b_r, W_e, att, bias):
    f32 = jnp.float32
    i32 = jnp.int32
    src = edge_index[0].astype(i32)
    dst = edge_index[1].astype(i32)
    arN = jnp.arange(NP, dtype=i32)
    zpad = jnp.zeros((PAD1,), i32)
    dpad = jnp.full((PAD1,), DUMMY, i32)

    src_g = jnp.concatenate([src, zpad, arN])
    dst_g = jnp.concatenate([dst, zpad, arN])
    dst_w = jnp.concatenate([dst, dpad, jnp.full((NP,), DUMMY, i32)])
    dst_s = jnp.concatenate([dst, dpad, arN])
    ew_t = jnp.concatenate(
        [edge_weight.astype(f32), jnp.zeros((PAD1 + NP,), f32)])

    x_p = jnp.pad(x.astype(f32), ((0, NP - N), (0, 0)))
    bl2 = b_l.reshape(1, C).astype(f32)
    br2 = b_r.reshape(1, C).astype(f32)
    we2 = W_e.reshape(1, C).astype(f32)
    att2 = att.reshape(1, C).astype(f32)
    bias2 = bias.reshape(1, C).astype(f32)

    xl, xr = pl.pallas_call(
        _proj_body,
        grid=(NNBLK,),
        in_specs=[
            pl.BlockSpec((NBLK, IN_CH), lambda i: (i, 0)),
            pl.BlockSpec((IN_CH, C), lambda i: (0, 0)),
            pl.BlockSpec((1, C), lambda i: (0, 0)),
            pl.BlockSpec((IN_CH, C), lambda i: (0, 0)),
            pl.BlockSpec((1, C), lambda i: (0, 0)),
        ],
        out_specs=[
            pl.BlockSpec((NBLK, C), lambda i: (i, 0)),
            pl.BlockSpec((NBLK, C), lambda i: (i, 0)),
        ],
        out_shape=[
            jax.ShapeDtypeStruct((NP, C), f32),
            jax.ShapeDtypeStruct((NP, C), f32),
        ],
    )(x_p, W_l.astype(f32), bl2, W_r.astype(f32), br2)

    XL, XR, WACC = _sc_gather_wstats(src_g, dst_g, dst_w, ew_t, xl, xr)

    la = pl.pallas_call(
        _la_body,
        grid=(NNBLK,),
        in_specs=[pl.BlockSpec((2, NBLK, 16), lambda i: (0, i, 0))],
        out_specs=pl.BlockSpec((NBLK, 1), lambda i: (i, 0)),
        out_shape=jax.ShapeDtypeStruct((NP, 1), f32),
    )(WACC)

    attr = jnp.concatenate([ew_t[:E + PAD1], la[:, 0]]).reshape(ET, 1)

    alpha, pmax = pl.pallas_call(
        _alpha_body,
        grid=(NEBLK,),
        in_specs=[
            pl.BlockSpec((EBLK, C), lambda i: (i, 0)),
            pl.BlockSpec((EBLK, C), lambda i: (i, 0)),
            pl.BlockSpec((EBLK, 1), lambda i: (i, 0)),
            pl.BlockSpec((1, C), lambda i: (0, 0)),
            pl.BlockSpec((1, C), lambda i: (0, 0)),
        ],
        out_specs=[
            pl.BlockSpec((EBLK, 1), lambda i: (i, 0)),
            pl.BlockSpec((1, 1, 128), lambda i: (i, 0, 0)),
        ],
        out_shape=[
            jax.ShapeDtypeStruct((ET, 1), f32),
            jax.ShapeDtypeStruct((NEBLK, 1, 128), f32),
        ],
    )(XL, XR, attr, we2, att2)

    gmax = jnp.max(pmax).reshape(1, 1)

    msg = pl.pallas_call(
        _msg_body,
        grid=(NEBLK,),
        in_specs=[
            pl.BlockSpec((EBLK, 1), lambda i: (i, 0)),
            pl.BlockSpec((EBLK, C), lambda i: (i, 0)),
            pl.BlockSpec((1, 1), lambda i: (0, 0)),
        ],
        out_specs=pl.BlockSpec((EBLK, 48), lambda i: (i, 0)),
        out_shape=jax.ShapeDtypeStruct((ET, 48), f32),
    )(alpha, XL, gmax)

    ACC = _sc_scatter_msgs(msg, dst_s)

    out = pl.pallas_call(
        _final_body,
        grid=(NNBLK,),
        in_specs=[
            pl.BlockSpec((2, NBLK, 48), lambda i: (0, i, 0)),
            pl.BlockSpec((1, C), lambda i: (0, 0)),
        ],
        out_specs=pl.BlockSpec((NBLK, C), lambda i: (i, 0)),
        out_shape=jax.ShapeDtypeStruct((NP, C), f32),
    )(ACC, bias2)

    return out[:N]


# speedup vs baseline: 4.7699x; 4.7699x over previous
import functools

import jax
import jax.numpy as jnp
from jax import lax
from jax.experimental import pallas as pl
from jax.experimental.pallas import tpu as pltpu
from jax.experimental.pallas import tpu_sc as plsc

N = 10000
NP = 10240
E = 320000
PAD1 = 1536
ET = E + PAD1 + NP
IN_CH = 128
C = 32
DUMMY = N

NWORK = 32
EPW = ET // NWORK
CHUNK = 128
NCH = EPW // CHUNK
ZROWS = NP // 16

EBLK = 2048
NEBLK = ET // EBLK
NBLK = 1280
NNBLK = NP // NBLK

_mesh = plsc.VectorSubcoreMesh(core_axis_name="c", subcore_axis_name="s")


@functools.partial(
    pl.kernel,
    mesh=_mesh,
    out_type=[
        jax.ShapeDtypeStruct((ET, C), jnp.float32),
        jax.ShapeDtypeStruct((ET, C), jnp.float32),
        jax.ShapeDtypeStruct((2, NP, 16), jnp.float32),
    ],
    scratch_types=[
        pltpu.VMEM((CHUNK,), jnp.int32),
        pltpu.VMEM((CHUNK,), jnp.int32),
        pltpu.VMEM((CHUNK,), jnp.int32),
        pltpu.VMEM((CHUNK, 16), jnp.float32),
        pltpu.VMEM((CHUNK, C), jnp.float32),
        pltpu.VMEM((CHUNK, C), jnp.float32),
        pltpu.VMEM((ZROWS, 16), jnp.float32),
        pltpu.VMEM_SHARED((NP, 16), jnp.float32),
        pltpu.SemaphoreType.DMA,
        pltpu.SemaphoreType.DMA,
    ],
    compiler_params=pltpu.CompilerParams(use_tc_tiling_on_sc=False),
)
def _sc_gather_wstats(srcg_hbm, dstg_hbm, dstw_hbm, ew16_hbm, xl_hbm, xr_hbm,
                      XL_hbm, XR_hbm, WACC_hbm,
                      isrc, idstg, idstw, w16, rows_l, rows_r, zbuf,
                      wacc_sh, sem1, sem2):
    cid = lax.axis_index("c")
    sid = lax.axis_index("s")
    wid = sid * 2 + cid
    zero16 = jnp.zeros((16,), jnp.float32)

    @pl.loop(0, ZROWS)
    def _(r):
        zbuf[r, :] = zero16

    pltpu.sync_copy(zbuf, wacc_sh.at[pl.ds(sid * ZROWS, ZROWS)])
    plsc.subcore_barrier()

    @pl.loop(0, NCH)
    def _(j):
        base = wid * EPW + j * CHUNK
        pltpu.sync_copy(srcg_hbm.at[pl.ds(base, CHUNK)], isrc)
        pltpu.sync_copy(dstg_hbm.at[pl.ds(base, CHUNK)], idstg)
        pltpu.sync_copy(dstw_hbm.at[pl.ds(base, CHUNK)], idstw)
        pltpu.sync_copy(ew16_hbm.at[pl.ds(base, CHUNK)], w16)
        cl = pltpu.async_copy(xl_hbm.at[isrc], rows_l, sem1)
        cr = pltpu.async_copy(xr_hbm.at[idstg], rows_r, sem2)
        pltpu.sync_copy(w16, wacc_sh.at[idstw], add=True)
        cl.wait()
        cr.wait()
        pltpu.sync_copy(rows_l, XL_hbm.at[pl.ds(base, CHUNK)])
        pltpu.sync_copy(rows_r, XR_hbm.at[pl.ds(base, CHUNK)])

    plsc.subcore_barrier()
    pltpu.sync_copy(wacc_sh.at[pl.ds(sid * ZROWS, ZROWS)],
                    WACC_hbm.at[cid, pl.ds(sid * ZROWS, ZROWS)])


@functools.partial(
    pl.kernel,
    mesh=_mesh,
    out_type=jax.ShapeDtypeStruct((2, NP, 48), jnp.float32),
    scratch_types=[
        pltpu.VMEM((CHUNK,), jnp.int32),
        pltpu.VMEM((CHUNK, 48), jnp.float32),
        pltpu.VMEM((ZROWS, 48), jnp.float32),
        pltpu.VMEM_SHARED((NP, 48), jnp.float32),
        pltpu.SemaphoreType.DMA,
    ],
    compiler_params=pltpu.CompilerParams(use_tc_tiling_on_sc=False),
)
def _sc_scatter_msgs(msg_hbm, dsts_hbm, ACC_hbm, idx, mbuf, zbuf, acc_sh, sem):
    cid = lax.axis_index("c")
    sid = lax.axis_index("s")
    wid = sid * 2 + cid
    zero16 = jnp.zeros((16,), jnp.float32)

    @pl.loop(0, ZROWS)
    def _(r):
        for k in range(3):
            zbuf[r, pl.ds(k * 16, 16)] = zero16

    pltpu.sync_copy(zbuf, acc_sh.at[pl.ds(sid * ZROWS, ZROWS)])
    plsc.subcore_barrier()

    @pl.loop(0, NCH)
    def _(j):
        base = wid * EPW + j * CHUNK
        pltpu.sync_copy(dsts_hbm.at[pl.ds(base, CHUNK)], idx)
        pltpu.sync_copy(msg_hbm.at[pl.ds(base, CHUNK)], mbuf)
        pltpu.sync_copy(mbuf, acc_sh.at[idx], add=True)

    plsc.subcore_barrier()
    pltpu.sync_copy(acc_sh.at[pl.ds(sid * ZROWS, ZROWS)],
                    ACC_hbm.at[cid, pl.ds(sid * ZROWS, ZROWS)])


def _proj_body(x_ref, wl_ref, bl_ref, wr_ref, br_ref, xl_ref, xr_ref):
    xb = x_ref[...]
    xl_ref[...] = jnp.dot(xb, wl_ref[...],
                          preferred_element_type=jnp.float32) + bl_ref[...]
    xr_ref[...] = jnp.dot(xb, wr_ref[...],
                          preferred_element_type=jnp.float32) + br_ref[...]


def _ew16_body(ew_ref, ew16_ref):
    e = ew_ref[...]
    ew16_ref[...] = jnp.concatenate(
        [e, jnp.ones((EBLK, 1), jnp.float32),
         jnp.zeros((EBLK, 14), jnp.float32)], axis=1)


def _la_body(wacc_ref, la_ref):
    w = wacc_ref[0] + wacc_ref[1]
    la_ref[...] = w[:, 0:1] / jnp.maximum(w[:, 1:2], 1.0)


def _alpha_body(xl_ref, xr_ref, attr_ref, we_ref, att_ref, alpha_ref, pmax_ref):
    m = xl_ref[...] + xr_ref[...] + attr_ref[...] * we_ref[...]
    m = jnp.where(m >= 0.0, m, 0.2 * m)
    a = jnp.sum(m * att_ref[...], axis=1, keepdims=True)
    alpha_ref[...] = a
    pmax_ref[...] = jnp.broadcast_to(jnp.max(a), (1, 1, 128))


def _msg_body(alpha_ref, xl_ref, g_ref, msg_ref):
    ex = jnp.exp(alpha_ref[...] - g_ref[...])
    msg_ref[...] = jnp.concatenate(
        [ex * xl_ref[...], jnp.broadcast_to(ex, (EBLK, 16))], axis=1)


def _final_body(acc_ref, bias_ref, out_ref):
    a = acc_ref[0] + acc_ref[1]
    o = a[:, :C] / (a[:, C:C + 1] + 1e-16) + bias_ref[...]
    out_ref[...] = jnp.where(o >= 0.0, o, 0.01 * o)


def kernel(x, edge_index, edge_weight, W_l, b_l, W_r, b_r, W_e, att, bias):
    f32 = jnp.float32
    i32 = jnp.int32
    src = edge_index[0].astype(i32)
    dst = edge_index[1].astype(i32)
    arN = jnp.arange(NP, dtype=i32)
    zpad = jnp.zeros((PAD1,), i32)
    dpad = jnp.full((PAD1,), DUMMY, i32)

    src_g = jnp.concatenate([src, zpad, arN])
    dst_g = jnp.concatenate([dst, zpad, arN])
    dst_w = jnp.concatenate([dst, dpad, jnp.full((NP,), DUMMY, i32)])
    dst_s = jnp.concatenate([dst, dpad, arN])
    ew_t = jnp.concatenate(
        [edge_weight.astype(f32), jnp.zeros((PAD1 + NP,), f32)])

    x_p = jnp.pad(x.astype(f32), ((0, NP - N), (0, 0)))
    bl2 = b_l.reshape(1, C).astype(f32)
    br2 = b_r.reshape(1, C).astype(f32)
    we2 = W_e.reshape(1, C).astype(f32)
    att2 = att.reshape(1, C).astype(f32)
    bias2 = bias.reshape(1, C).astype(f32)

    xl, xr = pl.pallas_call(
        _proj_body,
        grid=(NNBLK,),
        in_specs=[
            pl.BlockSpec((NBLK, IN_CH), lambda i: (i, 0)),
            pl.BlockSpec((IN_CH, C), lambda i: (0, 0)),
            pl.BlockSpec((1, C), lambda i: (0, 0)),
            pl.BlockSpec((IN_CH, C), lambda i: (0, 0)),
            pl.BlockSpec((1, C), lambda i: (0, 0)),
        ],
        out_specs=[
            pl.BlockSpec((NBLK, C), lambda i: (i, 0)),
            pl.BlockSpec((NBLK, C), lambda i: (i, 0)),
        ],
        out_shape=[
            jax.ShapeDtypeStruct((NP, C), f32),
            jax.ShapeDtypeStruct((NP, C), f32),
        ],
    )(x_p, W_l.astype(f32), bl2, W_r.astype(f32), br2)

    ew16 = pl.pallas_call(
        _ew16_body,
        grid=(NEBLK,),
        in_specs=[pl.BlockSpec((EBLK, 1), lambda i: (i, 0))],
        out_specs=pl.BlockSpec((EBLK, 16), lambda i: (i, 0)),
        out_shape=jax.ShapeDtypeStruct((ET, 16), f32),
    )(ew_t.reshape(ET, 1))
    XL, XR, WACC = _sc_gather_wstats(src_g, dst_g, dst_w, ew16, xl, xr)

    la = pl.pallas_call(
        _la_body,
        grid=(NNBLK,),
        in_specs=[pl.BlockSpec((2, NBLK, 16), lambda i: (0, i, 0))],
        out_specs=pl.BlockSpec((NBLK, 1), lambda i: (i, 0)),
        out_shape=jax.ShapeDtypeStruct((NP, 1), f32),
    )(WACC)

    attr = jnp.concatenate([ew_t[:E + PAD1], la[:, 0]]).reshape(ET, 1)

    alpha, pmax = pl.pallas_call(
        _alpha_body,
        grid=(NEBLK,),
        in_specs=[
            pl.BlockSpec((EBLK, C), lambda i: (i, 0)),
            pl.BlockSpec((EBLK, C), lambda i: (i, 0)),
            pl.BlockSpec((EBLK, 1), lambda i: (i, 0)),
            pl.BlockSpec((1, C), lambda i: (0, 0)),
            pl.BlockSpec((1, C), lambda i: (0, 0)),
        ],
        out_specs=[
            pl.BlockSpec((EBLK, 1), lambda i: (i, 0)),
            pl.BlockSpec((1, 1, 128), lambda i: (i, 0, 0)),
        ],
        out_shape=[
            jax.ShapeDtypeStruct((ET, 1), f32),
            jax.ShapeDtypeStruct((NEBLK, 1, 128), f32),
        ],
    )(XL, XR, attr, we2, att2)

    gmax = jnp.max(pmax).reshape(1, 1)

    msg = pl.pallas_call(
        _msg_body,
        grid=(NEBLK,),
        in_specs=[
            pl.BlockSpec((EBLK, 1), lambda i: (i, 0)),
            pl.BlockSpec((EBLK, C), lambda i: (i, 0)),
            pl.BlockSpec((1, 1), lambda i: (0, 0)),
        ],
        out_specs=pl.BlockSpec((EBLK, 48), lambda i: (i, 0)),
        out_shape=jax.ShapeDtypeStruct((ET, 48), f32),
    )(alpha, XL, gmax)

    ACC = _sc_scatter_msgs(msg, dst_s)

    out = pl.pallas_call(
        _final_body,
        grid=(NNBLK,),
        in_specs=[
            pl.BlockSpec((2, NBLK, 48), lambda i: (0, i, 0)),
            pl.BlockSpec((1, C), lambda i: (0, 0)),
        ],
        out_specs=pl.BlockSpec((NBLK, C), lambda i: (i, 0)),
        out_shape=jax.ShapeDtypeStruct((NP, C), f32),
    )(ACC, bias2)

    return out[:N]
